# baseline (speedup 1.0000x reference)
import functools

import jax
import jax.numpy as jnp
from jax.experimental import pallas as pl
from jax.experimental.pallas import tpu as pltpu


def _round_up(x: int, m: int) -> int:
    return ((x + m - 1) // m) * m


def _xw_kernel(x_ref, w_ref, o_ref):
    o_ref[...] = jnp.dot(
        x_ref[...].astype(jnp.bfloat16),
        w_ref[...].astype(jnp.bfloat16),
        preferred_element_type=jnp.float32,
    ).astype(o_ref.dtype)


def _transform_features(x_pad, w_pad, tile_rows):
    n_pad, in_f = x_pad.shape
    out_pad = w_pad.shape[1]
    grid = (n_pad // tile_rows,)
    return pl.pallas_call(
        _xw_kernel,
        out_shape=jax.ShapeDtypeStruct((n_pad, out_pad), jnp.bfloat16),
        grid_spec=pltpu.PrefetchScalarGridSpec(
            num_scalar_prefetch=0,
            grid=grid,
            in_specs=[
                pl.BlockSpec((tile_rows, in_f), lambda i: (i, 0)),
                pl.BlockSpec((in_f, out_pad), lambda i: (0, 0)),
            ],
            out_specs=pl.BlockSpec((tile_rows, out_pad), lambda i: (i, 0)),
        ),
        compiler_params=pltpu.CompilerParams(
            dimension_semantics=("parallel",),
        ),
        cost_estimate=pl.CostEstimate(
            flops=2 * n_pad * in_f * out_pad,
            transcendentals=0,
            bytes_accessed=4 * (n_pad * in_f + in_f * out_pad)
            + 2 * n_pad * out_pad,
        ),
    )(x_pad, w_pad)


def _agg_kernel(a_ref, xw_ref, o_ref):
    o_ref[...] = jnp.dot(
        a_ref[...].astype(jnp.bfloat16),
        xw_ref[...],
        preferred_element_type=jnp.float32,
    )


@functools.partial(jax.jit, static_argnames=("tile_m",))
def _gcn_fixed_w(W, x, A, tile_m=128):
    n_rows, n_cols = A.shape
    n_nodes, in_f = x.shape
    out_f = W.shape[1]
    assert W.shape[0] == in_f
    assert n_cols == n_nodes
    out_dtype = x.dtype

    out_pad = _round_up(out_f, 128)
    tile_m = min(tile_m, _round_up(n_rows, 8))
    n_rows_pad = _round_up(n_rows, tile_m)
    n_cols_pad = _round_up(n_cols, 128)

    A_pad = jnp.pad(A.astype(jnp.float32),
                    ((0, n_rows_pad - n_rows), (0, n_cols_pad - n_cols)))
    x_pad = jnp.pad(x.astype(jnp.float32),
                    ((0, n_cols_pad - n_nodes), (0, 0)))
    W_pad = jnp.pad(W.astype(jnp.float32),
                    ((0, 0), (0, out_pad - out_f)))

    xw = _transform_features(x_pad, W_pad, n_cols_pad // 2)

    grid = (n_rows_pad // tile_m,)
    out_padded = pl.pallas_call(
        _agg_kernel,
        out_shape=jax.ShapeDtypeStruct((n_rows_pad, out_pad), jnp.float32),
        grid_spec=pltpu.PrefetchScalarGridSpec(
            num_scalar_prefetch=0,
            grid=grid,
            in_specs=[
                pl.BlockSpec((tile_m, n_cols_pad), lambda i: (i, 0)),
                pl.BlockSpec((n_cols_pad, out_pad), lambda i: (0, 0)),
            ],
            out_specs=pl.BlockSpec((tile_m, out_pad), lambda i: (i, 0)),
        ),
        compiler_params=pltpu.CompilerParams(
            dimension_semantics=("parallel",),
        ),
        cost_estimate=pl.CostEstimate(
            flops=2 * n_rows_pad * n_cols_pad * out_pad,
            transcendentals=0,
            bytes_accessed=4 * (n_rows_pad * n_cols_pad + n_rows_pad * out_pad)
            + 2 * n_cols_pad * out_pad,
        ),
    )(A_pad, xw)

    return out_padded[:n_rows, :out_f].astype(out_dtype)


def kernel(W, x, A):
    return _gcn_fixed_w(W, x, A)


# speedup vs baseline: 1.4859x; 1.4859x over previous
import functools

import jax
import jax.numpy as jnp
from jax.experimental import pallas as pl
from jax.experimental.pallas import tpu as pltpu


def _round_up(x: int, m: int) -> int:
    return ((x + m - 1) // m) * m


def _xw_kernel(x_ref, w_ref, o_ref):
    o_ref[...] = jnp.dot(
        x_ref[...].astype(jnp.bfloat16),
        w_ref[...].astype(jnp.bfloat16),
        preferred_element_type=jnp.float32,
    ).astype(o_ref.dtype)


def _transform_features(x_pad, w_pad, tile_rows):
    n_pad, in_f = x_pad.shape
    out_pad = w_pad.shape[1]
    grid = (n_pad // tile_rows,)
    return pl.pallas_call(
        _xw_kernel,
        out_shape=jax.ShapeDtypeStruct((n_pad, out_pad), jnp.bfloat16),
        grid_spec=pltpu.PrefetchScalarGridSpec(
            num_scalar_prefetch=0,
            grid=grid,
            in_specs=[
                pl.BlockSpec((tile_rows, in_f), lambda i: (i, 0)),
                pl.BlockSpec((in_f, out_pad), lambda i: (0, 0)),
            ],
            out_specs=pl.BlockSpec((tile_rows, out_pad), lambda i: (i, 0)),
        ),
        compiler_params=pltpu.CompilerParams(
            dimension_semantics=("parallel",),
        ),
        cost_estimate=pl.CostEstimate(
            flops=2 * n_pad * in_f * out_pad,
            transcendentals=0,
            bytes_accessed=4 * (n_pad * in_f + in_f * out_pad)
            + 2 * n_pad * out_pad,
        ),
    )(x_pad, w_pad)


def _agg_kernel(a_ref, xw_ref, o_ref):
    o_ref[...] = jnp.dot(
        a_ref[...].astype(jnp.bfloat16),
        xw_ref[...],
        preferred_element_type=jnp.float32,
    )


@functools.partial(jax.jit, static_argnames=("tile_m",))
def _gcn_fixed_w(W, x, A, tile_m=512):
    n_rows, n_cols = A.shape
    n_nodes, in_f = x.shape
    out_f = W.shape[1]
    assert W.shape[0] == in_f
    assert n_cols == n_nodes
    out_dtype = x.dtype

    out_pad = _round_up(out_f, 128)
    tile_m = min(tile_m, _round_up(n_rows, 8))
    n_rows_pad = _round_up(n_rows, tile_m)
    n_cols_pad = _round_up(n_cols, 128)

    A_pad = jnp.pad(A.astype(jnp.float32),
                    ((0, n_rows_pad - n_rows), (0, n_cols_pad - n_cols)))
    x_pad = jnp.pad(x.astype(jnp.float32),
                    ((0, n_cols_pad - n_nodes), (0, 0)))
    W_pad = jnp.pad(W.astype(jnp.float32),
                    ((0, 0), (0, out_pad - out_f)))

    xw = _transform_features(x_pad, W_pad, n_cols_pad // 2)

    grid = (n_rows_pad // tile_m,)
    out_padded = pl.pallas_call(
        _agg_kernel,
        out_shape=jax.ShapeDtypeStruct((n_rows_pad, out_pad), jnp.float32),
        grid_spec=pltpu.PrefetchScalarGridSpec(
            num_scalar_prefetch=0,
            grid=grid,
            in_specs=[
                pl.BlockSpec((tile_m, n_cols_pad), lambda i: (i, 0)),
                pl.BlockSpec((n_cols_pad, out_pad), lambda i: (0, 0)),
            ],
            out_specs=pl.BlockSpec((tile_m, out_pad), lambda i: (i, 0)),
        ),
        compiler_params=pltpu.CompilerParams(
            dimension_semantics=("parallel",),
        ),
        cost_estimate=pl.CostEstimate(
            flops=2 * n_rows_pad * n_cols_pad * out_pad,
            transcendentals=0,
            bytes_accessed=4 * (n_rows_pad * n_cols_pad + n_rows_pad * out_pad)
            + 2 * n_cols_pad * out_pad,
        ),
    )(A_pad, xw)

    return out_padded[:n_rows, :out_f].astype(out_dtype)


def kernel(W, x, A):
    return _gcn_fixed_w(W, x, A)
